# baseline (speedup 1.0000x reference)
import functools

import jax
import jax.numpy as jnp
from jax import lax
from jax.experimental import pallas as pl
from jax.experimental.pallas import tpu as pltpu
from jax.experimental.pallas import tpu_sc as plsc

N = 10000
E = 320000
NP = 10240
B = 128
Q = 1
NB = 80
EPT = NB * Q * B
EPAD = 32 * EPT
SLICE = NP // 16

_SELU_SCALE = 1.0507009873554805
_SELU_ALPHA = 1.6732632423543772


def _selu(x):
    return _SELU_SCALE * jnp.where(x > 0, x, _SELU_ALPHA * (jnp.exp(x) - 1.0))


def _zero_rows(rows, nrow, w):
    def zr(i, _):
        def zc(c, _):
            rows[i, pl.ds(c * 16, 16)] = jnp.zeros((16,), jnp.float32)
            return 0
        return lax.fori_loop(0, w // 16, zc, 0)
    lax.fori_loop(0, nrow, zr, 0)


def _zero_vec(buf, n):
    def zb(i, _):
        buf[pl.ds(i * 16, 16)] = jnp.zeros((16,), jnp.float32)
        return 0
    lax.fori_loop(0, n // 16, zb, 0)


def _gat_body(W, splits, with_deg, srcp, dstp, a_s_h, a_d_h, *rest):
    Wc = W // splits
    tables = rest[:splits]
    rest = rest[splits:]
    out_h = rest[0]
    rest = rest[1:]
    if with_deg:
        deg_h = rest[0]
        rest = rest[1:]
    (a_s, a_d, src_l, dst_l, ee_l, dn_l, rows, zbuf, ones_b,
     acc, dn_acc) = rest[:11]
    rest = rest[11:]
    if with_deg:
        deg_acc = rest[0]
        rest = rest[1:]
    sem, sem2 = rest[0], rest[1]

    cid = lax.axis_index("c")
    sid = lax.axis_index("s")

    _zero_vec(zbuf, SLICE)
    pltpu.sync_copy(zbuf, dn_acc.at[pl.ds(sid * SLICE, SLICE)])
    if with_deg:
        pltpu.sync_copy(zbuf, deg_acc.at[pl.ds(sid * SLICE, SLICE)])

    pltpu.sync_copy(a_s_h, a_s)
    pltpu.sync_copy(a_d_h, a_d)
    def one16(i, _):
        ones_b[pl.ds(i * 16, 16)] = jnp.ones((16,), jnp.float32)
        return 0
    lax.fori_loop(0, B // 16, one16, 0)

    plsc.subcore_barrier()

    for t in range(2):
        chunk = sid * 2 + jnp.where(jnp.int32(t) == 0, 1 - cid, cid)
        pltpu.sync_copy(srcp.at[chunk], src_l)
        pltpu.sync_copy(dstp.at[chunk], dst_l)

        def logits(j, _):
            def inner(k, _):
                q = k // (B // 16)
                kk = k % (B // 16)
                sl = pl.ds(kk * 16, 16)
                sv = src_l[j, q, sl]
                dv = dst_l[j, q, sl]
                av = plsc.load_gather(a_s, [sv])
                bv = plsc.load_gather(a_d, [dv])
                e = av + bv
                e = jnp.where(e > 0, e, 0.2 * e)
                ee_l[j, q, sl] = jnp.exp(e)
                return 0
            return lax.fori_loop(0, Q * B // 16, inner, 0)
        lax.fori_loop(0, NB, logits, 0)

        descs = []
        for j in range(NB):
            descs.append(pltpu.async_copy(
                ee_l.at[j, 0], dn_acc.at[dst_l.at[j, 0]], sem, add=True))
            if with_deg:
                descs.append(pltpu.async_copy(
                    ones_b, deg_acc.at[src_l.at[j, 0]], sem, add=True))
        for d in descs:
            d.wait()

    plsc.subcore_barrier()

    pltpu.sync_copy(dn_acc, dn_l)

    def alphas(j, _):
        def inner(k, _):
            q = k // (B // 16)
            kk = k % (B // 16)
            sl = pl.ds(kk * 16, 16)
            dv = dst_l[j, q, sl]
            dn = plsc.load_gather(dn_l, [dv])
            ee_l[j, q, sl] = ee_l[j, q, sl] / (dn + 1e-16)
            return 0
        return lax.fori_loop(0, Q * B // 16, inner, 0)
    lax.fori_loop(0, NB, alphas, 0)

    GK = 2
    for half, table_h in enumerate(tables):
        _zero_rows(rows.at[0, 0], B, Wc)
        for q in range(SLICE // B):
            pltpu.sync_copy(rows.at[0, 0],
                            acc.at[pl.ds(sid * SLICE + q * B, B)])
        plsc.subcore_barrier()

        def spmm(g, _):
            descs = [pltpu.async_copy(
                table_h.at[src_l.at[g * GK + k, 0]], rows.at[k, 0], sem)
                for k in range(GK)]
            for d in descs:
                d.wait()
            for k in range(GK):
                j = g * GK + k

                def scale(i, _, k=k, j=j):
                    q = i // (B // 16)
                    kk = i % (B // 16)
                    a16 = ee_l[j, q, pl.ds(kk * 16, 16)]
                    for l in range(16):
                        a = a16[l]
                        r = kk * 16 + l
                        for c in range(Wc // 16):
                            sl = pl.ds(c * 16, 16)
                            rows[k, q, r, sl] = rows[k, q, r, sl] * a
                    return 0
                lax.fori_loop(0, Q * B // 16, scale, 0)
            descs = [pltpu.async_copy(
                rows.at[k, 0], acc.at[dst_l.at[g * GK + k, 0]], sem2, add=True)
                for k in range(GK)]
            for d in descs:
                d.wait()
            return 0
        lax.fori_loop(0, NB // GK, spmm, 0)

        plsc.subcore_barrier()

        sl = pl.ds(sid * SLICE, SLICE)
        if splits == 1:
            pltpu.sync_copy(acc.at[sl], out_h.at[cid, sl])
        else:
            pltpu.sync_copy(acc.at[sl], out_h.at[cid, half, sl])

    if with_deg:
        sl = pl.ds(sid * SLICE, SLICE)
        @pl.when(cid == 0)
        def _():
            pltpu.sync_copy(deg_acc.at[sl], deg_h.at[sl])


def _sc_gat(W, with_deg):
    splits = 4 if W > 64 else 1
    Wc = W // splits
    mesh = plsc.VectorSubcoreMesh(core_axis_name="c", subcore_axis_name="s")
    out_shape = (2, NP, W) if splits == 1 else (2, splits, NP, Wc)
    out_type = [jax.ShapeDtypeStruct(out_shape, jnp.float32)]
    if with_deg:
        out_type.append(jax.ShapeDtypeStruct((NP,), jnp.float32))
    scratch = [
        pltpu.VMEM((NP,), jnp.float32),
        pltpu.VMEM((NP,), jnp.float32),
        pltpu.VMEM((NB, Q, B), jnp.int32),
        pltpu.VMEM((NB, Q, B), jnp.int32),
        pltpu.VMEM((NB, Q, B), jnp.float32),
        pltpu.VMEM((NP,), jnp.float32),
        pltpu.VMEM((2, Q, B, Wc), jnp.float32),
        pltpu.VMEM((SLICE,), jnp.float32),
        pltpu.VMEM((B,), jnp.float32),
        pltpu.VMEM_SHARED((NP, Wc), jnp.float32),
        pltpu.VMEM_SHARED((NP,), jnp.float32),
    ]
    if with_deg:
        scratch.append(pltpu.VMEM_SHARED((NP,), jnp.float32))
    scratch.append(pltpu.SemaphoreType.DMA)
    scratch.append(pltpu.SemaphoreType.DMA)
    return pl.kernel(
        functools.partial(_gat_body, W, splits, with_deg),
        out_type=tuple(out_type) if len(out_type) > 1 else out_type[0],
        mesh=mesh,
        scratch_types=tuple(scratch),
        compiler_params=pltpu.CompilerParams(needs_layout_passes=False, use_tc_tiling_on_sc=False),
    )


def _smooth_body(n_out, colp, rowp, ta_h, tb_h, *args):
    t1_h = args[0]
    args = args[1:]
    if n_out == 2:
        t2_h = args[0]
        args = args[1:]
    out1_h = args[0]
    args = args[1:]
    if n_out == 2:
        out2_h = args[0]
        args = args[1:]
    (ta, tb, col_l, row_l, w1_l, rows1) = args[:6]
    args = args[6:]
    if n_out == 2:
        acc2 = args[0]
        args = args[1:]
    acc1 = args[0]
    sem, sem2 = args[1], args[2]

    cid = lax.axis_index("c")
    sid = lax.axis_index("s")
    wid = sid * 2 + cid

    Wd = 32
    _zero_rows(rows1.at[0, 0], B, Wd)
    for q in range(SLICE // B):
        sl = pl.ds(sid * SLICE + q * B, B)
        pltpu.sync_copy(rows1.at[0, 0], acc1.at[sl])
        if n_out == 2:
            pltpu.sync_copy(rows1.at[0, 0], acc2.at[sl])

    pltpu.sync_copy(ta_h, ta)
    pltpu.sync_copy(tb_h, tb)
    pltpu.sync_copy(colp.at[wid], col_l)
    pltpu.sync_copy(rowp.at[wid], row_l)

    def weights(j, _):
        def inner(k, _):
            q = k // (B // 16)
            kk = k % (B // 16)
            sl = pl.ds(kk * 16, 16)
            rv = row_l[j, q, sl]
            cv = col_l[j, q, sl]
            tbv = plsc.load_gather(tb, [cv])
            w1_l[j, q, sl] = plsc.load_gather(ta, [rv]) * tbv
            return 0
        return lax.fori_loop(0, Q * B // 16, inner, 0)
    lax.fori_loop(0, NB, weights, 0)

    plsc.subcore_barrier()

    GK = 2
    passes = [(t1_h, acc1)] if n_out == 1 else [(t1_h, acc1), (t2_h, acc2)]

    def spmm(g, _):
        for tab_h, acc in passes:
            descs = [pltpu.async_copy(
                tab_h.at[col_l.at[g * GK + k, 0]], rows1.at[k, 0], sem)
                for k in range(GK)]
            for d in descs:
                d.wait()
            for k in range(GK):
                j = g * GK + k

                def scale(i, _, k=k, j=j):
                    q = i // (B // 16)
                    kk = i % (B // 16)
                    a16 = w1_l[j, q, pl.ds(kk * 16, 16)]
                    for l in range(16):
                        a1 = a16[l]
                        r = kk * 16 + l
                        for c in range(Wd // 16):
                            sl = pl.ds(c * 16, 16)
                            rows1[k, q, r, sl] = rows1[k, q, r, sl] * a1
                    return 0
                lax.fori_loop(0, Q * B // 16, scale, 0)
            descs = [pltpu.async_copy(
                rows1.at[k, 0], acc.at[row_l.at[g * GK + k, 0]], sem2, add=True)
                for k in range(GK)]
            for d in descs:
                d.wait()
        return 0
    lax.fori_loop(0, NB // GK, spmm, 0)

    plsc.subcore_barrier()

    sl = pl.ds(sid * SLICE, SLICE)
    pltpu.sync_copy(acc1.at[sl], out1_h.at[cid, sl])
    if n_out == 2:
        pltpu.sync_copy(acc2.at[sl], out2_h.at[cid, sl])


def _sc_smooth(n_out):
    mesh = plsc.VectorSubcoreMesh(core_axis_name="c", subcore_axis_name="s")
    Wd = 32
    outs = [jax.ShapeDtypeStruct((2, NP, Wd), jnp.float32)] * n_out
    scratch = [
        pltpu.VMEM((NP,), jnp.float32),
        pltpu.VMEM((NP,), jnp.float32),
        pltpu.VMEM((NB, Q, B), jnp.int32),
        pltpu.VMEM((NB, Q, B), jnp.int32),
        pltpu.VMEM((NB, Q, B), jnp.float32),
        pltpu.VMEM((2, Q, B, Wd), jnp.float32),
    ]
    if n_out == 2:
        scratch += [
            pltpu.VMEM_SHARED((NP, Wd), jnp.float32),
        ]
    scratch += [
        pltpu.VMEM_SHARED((NP, Wd), jnp.float32),
        pltpu.SemaphoreType.DMA,
        pltpu.SemaphoreType.DMA,
    ]
    return pl.kernel(
        functools.partial(_smooth_body, n_out),
        out_type=tuple(outs) if n_out == 2 else outs[0],
        mesh=mesh,
        scratch_types=tuple(scratch),
        compiler_params=pltpu.CompilerParams(needs_layout_passes=False, use_tc_tiling_on_sc=False),
    )


_BLK = 1024
_G = NP // _BLK


def _tc0_body(x_ref, v1_ref, o1_ref, o2_ref):
    x = x_ref[...]
    o1_ref[...] = jnp.sum(x * v1_ref[0][None, :], axis=1)
    o2_ref[...] = jnp.sum(x * v1_ref[1][None, :], axis=1)


def _tc0(x_pad, v1s, v1d):
    return pl.pallas_call(
        _tc0_body,
        out_shape=(jax.ShapeDtypeStruct((NP,), jnp.float32),
                   jax.ShapeDtypeStruct((NP,), jnp.float32)),
        grid=(_G,),
        in_specs=[pl.BlockSpec((_BLK, 128), lambda i: (i, 0)),
                  pl.BlockSpec((2, 128), lambda i: (0, 0))],
        out_specs=(pl.BlockSpec((_BLK,), lambda i: (i,)),
                   pl.BlockSpec((_BLK,), lambda i: (i,))),
    )(x_pad, jnp.stack([v1s, v1d]))


def _tc1_body(p_ref, w1_ref, w2p_ref, v2_ref, xs2_ref, a2s_ref, a2d_ref):
    m = jnp.concatenate([p_ref[0, q] + p_ref[1, q] for q in range(4)],
                        axis=1)
    h1 = _selu(jnp.dot(m, w1_ref[...], preferred_element_type=jnp.float32))
    xs2_ref[...] = jnp.dot(h1, w2p_ref[...], preferred_element_type=jnp.float32)
    a2s_ref[...] = jnp.sum(h1 * v2_ref[0][None, :], axis=1)
    a2d_ref[...] = jnp.sum(h1 * v2_ref[1][None, :], axis=1)


def _tc1(m1P, W1_src, W2p, v2s, v2d):
    return pl.pallas_call(
        _tc1_body,
        out_shape=(jax.ShapeDtypeStruct((NP, 32), jnp.float32),
                   jax.ShapeDtypeStruct((NP,), jnp.float32),
                   jax.ShapeDtypeStruct((NP,), jnp.float32)),
        grid=(_G,),
        in_specs=[pl.BlockSpec((2, 4, _BLK, 32), lambda i: (0, 0, i, 0)),
                  pl.BlockSpec((128, 512), lambda i: (0, 0)),
                  pl.BlockSpec((512, 32), lambda i: (0, 0)),
                  pl.BlockSpec((2, 512), lambda i: (0, 0))],
        out_specs=(pl.BlockSpec((_BLK, 32), lambda i: (i, 0)),
                   pl.BlockSpec((_BLK,), lambda i: (i,)),
                   pl.BlockSpec((_BLK,), lambda i: (i,))),
    )(m1P, W1_src, W2p, jnp.stack([v2s, v2d]))


def _tc2_body(p_ref, deg_ref, v3_ref, sa_ref, h2_ref, h2a_ref, h2b_ref,
              a3s_ref, a3d_ref, dis_ref, dis2ad_ref, dis2_ref):
    h2 = p_ref[0] + p_ref[1]
    h2_ref[...] = h2
    h2a_ref[...] = h2 * sa_ref[0, 0]
    h2b_ref[...] = h2 * (2.0 * sa_ref[0, 1])
    a3s_ref[...] = jnp.sum(h2 * v3_ref[0][None, :], axis=1)
    a3d_ref[...] = jnp.sum(h2 * v3_ref[1][None, :], axis=1)
    deg = deg_ref[...]
    logd = jnp.log(jnp.maximum(deg, 1e-30))
    dis_ref[...] = jnp.where(deg > 0, jnp.exp(-0.6 * logd), 0.0)
    dis2 = jnp.where(deg > 0, jnp.exp(-0.5 * logd), 0.0)
    dis2_ref[...] = dis2
    alpha_d = 1.0 / (1.0 + jnp.log(deg + 1.0))
    dis2ad_ref[...] = dis2 * alpha_d


def _tc2(h2P, deg, v3s, v3d, sa1, sa2):
    return pl.pallas_call(
        _tc2_body,
        out_shape=(jax.ShapeDtypeStruct((NP, 32), jnp.float32),
                   jax.ShapeDtypeStruct((NP, 32), jnp.float32),
                   jax.ShapeDtypeStruct((NP, 32), jnp.float32),
                   jax.ShapeDtypeStruct((NP,), jnp.float32),
                   jax.ShapeDtypeStruct((NP,), jnp.float32),
                   jax.ShapeDtypeStruct((NP,), jnp.float32),
                   jax.ShapeDtypeStruct((NP,), jnp.float32),
                   jax.ShapeDtypeStruct((NP,), jnp.float32)),
        grid=(_G,),
        in_specs=[pl.BlockSpec((2, _BLK, 32), lambda i: (0, i, 0)),
                  pl.BlockSpec((_BLK,), lambda i: (i,)),
                  pl.BlockSpec((2, 32), lambda i: (0, 0)),
                  pl.BlockSpec(memory_space=pltpu.SMEM)],
        out_specs=(pl.BlockSpec((_BLK, 32), lambda i: (i, 0)),
                   pl.BlockSpec((_BLK, 32), lambda i: (i, 0)),
                   pl.BlockSpec((_BLK, 32), lambda i: (i, 0)),
                   pl.BlockSpec((_BLK,), lambda i: (i,)),
                   pl.BlockSpec((_BLK,), lambda i: (i,)),
                   pl.BlockSpec((_BLK,), lambda i: (i,)),
                   pl.BlockSpec((_BLK,), lambda i: (i,)),
                   pl.BlockSpec((_BLK,), lambda i: (i,))),
    )(h2P, deg, jnp.stack([v3s, v3d]),
      jnp.stack([sa1, sa2]).reshape(1, 2))


def _tc3_body(p_ref, w2t_ref, w1t_ref, h4_ref):
    m = p_ref[0] + p_ref[1]
    h3 = _selu(jnp.dot(m, w2t_ref[...], preferred_element_type=jnp.float32))
    h4_ref[...] = jnp.dot(h3, w1t_ref[...], preferred_element_type=jnp.float32)


def _tc3(m3P, W2pT, W1T):
    return pl.pallas_call(
        _tc3_body,
        out_shape=jax.ShapeDtypeStruct((NP, 128), jnp.float32),
        grid=(_G,),
        in_specs=[pl.BlockSpec((2, _BLK, 32), lambda i: (0, i, 0)),
                  pl.BlockSpec((32, 512), lambda i: (0, 0)),
                  pl.BlockSpec((512, 128), lambda i: (0, 0))],
        out_specs=pl.BlockSpec((_BLK, 128), lambda i: (i, 0)),
    )(m3P, W2pT, W1T)


def _combine_body(a_ref, o_ref, *, k, scale):
    acc = a_ref[0]
    for i in range(1, k):
        acc = acc + a_ref[i]
    o_ref[...] = acc * scale


def _combine(parts, scale=1.0):
    k, np_, w = parts.shape
    return pl.pallas_call(
        functools.partial(_combine_body, k=k, scale=scale),
        out_shape=jax.ShapeDtypeStruct((np_, w), jnp.float32),
        grid=(np_ // _BLK,),
        in_specs=[pl.BlockSpec((k, _BLK, w), lambda i: (0, i, 0))],
        out_specs=pl.BlockSpec((_BLK, w), lambda i: (i, 0)),
    )(parts)


def _combine2s_body(p1_ref, p2_ref, s_ref, o1_ref, o2_ref):
    o1_ref[...] = (p1_ref[0] + p1_ref[1]) * s_ref[0, 0]
    o2_ref[...] = (p2_ref[0] + p2_ref[1]) * (2.0 * s_ref[0, 1])


def _combine2s(p1, p2, sa1, sa2):
    return pl.pallas_call(
        _combine2s_body,
        out_shape=(jax.ShapeDtypeStruct((NP, 32), jnp.float32),
                   jax.ShapeDtypeStruct((NP, 32), jnp.float32)),
        grid=(_G,),
        in_specs=[pl.BlockSpec((2, _BLK, 32), lambda i: (0, i, 0)),
                  pl.BlockSpec((2, _BLK, 32), lambda i: (0, i, 0)),
                  pl.BlockSpec(memory_space=pltpu.SMEM)],
        out_specs=(pl.BlockSpec((_BLK, 32), lambda i: (i, 0)),
                   pl.BlockSpec((_BLK, 32), lambda i: (i, 0))),
    )(p1, p2, jnp.stack([sa1, sa2]).reshape(1, 2))


def kernel(features, edge_index, W1_src, W1_dst, att1_src, att1_dst,
           W2_src, W2_dst, att2_src, att2_dst, att3_src, att3_dst,
           smooth_alpha_1, smooth_alpha_2):
    f32 = jnp.float32
    src = edge_index[0]
    dst = edge_index[1]
    pad_e = EPAD - E
    srcp = jnp.concatenate([src, jnp.full((pad_e,), N, jnp.int32)]
                           ).reshape(32, NB, Q, B)
    dstp = jnp.concatenate([dst, jnp.full((pad_e,), N, jnp.int32)]
                           ).reshape(32, NB, Q, B)

    x_pad = jnp.pad(features, ((0, NP - N), (0, 0)))
    W2p = jnp.pad(W2_src, ((0, 0), (0, 2)))
    W2pT = jnp.pad(W2_src.T, ((0, 2), (0, 0)))
    W1T = W1_src.T
    v1s = W1_src @ att1_src
    v1d = W1_dst @ att1_dst
    v2s = W2_src @ att2_src
    v2d = W2_dst @ att2_dst
    v3s = jnp.pad(W2_src.T @ att3_src, (0, 2))
    v3d = jnp.pad(W2_dst.T @ att3_dst, (0, 2))

    a1s, a1d = _tc0(x_pad, v1s, v1d)
    m1P, deg = _sc_gat(128, True)(srcp, dstp, a1s, a1d,
                                  x_pad[:, :32], x_pad[:, 32:64],
                                  x_pad[:, 64:96], x_pad[:, 96:])
    xs2, a2s, a2d = _tc1(m1P, W1_src, W2p, v2s, v2d)

    h2P = _sc_gat(32, False)(srcp, dstp, a2s, a2d, xs2)
    h2, h2a, h2b, a3s, a3d, dis, dis2ad, dis2 = _tc2(
        h2P, deg, v3s, v3d, smooth_alpha_1, smooth_alpha_2)

    m3P = _sc_gat(32, False)(srcp, dstp, a3s, a3d, h2)
    h4 = _tc3(m3P, W2pT, W1T)

    f1P, f2P = _sc_smooth(2)(dstp, srcp, dis, dis, h2a, h2b)
    f1, f2 = _combine2s(f1P, f2P, smooth_alpha_1, smooth_alpha_2)
    f1P, f2P = _sc_smooth(2)(dstp, srcp, dis, dis, f1, f2)
    sm = _combine(jnp.concatenate([f1P, f2P]), 0.5)
    smP = _sc_smooth(1)(dstp, srcp, dis2ad, dis2, sm)
    sm = _combine(smP)
    smP = _sc_smooth(1)(dstp, srcp, dis2ad, dis2, sm)
    sm = _combine(smP)

    return (sm[:N, :30], h4[:N])


# speedup vs baseline: 1.0135x; 1.0135x over previous
import functools

import jax
import jax.numpy as jnp
from jax import lax
from jax.experimental import pallas as pl
from jax.experimental.pallas import tpu as pltpu
from jax.experimental.pallas import tpu_sc as plsc

N = 10000
E = 320000
NP = 10240
B = 128
NB = 80
EPT = NB * B
EPAD = 32 * EPT
SLICE = NP // 16
GK = 4

_SELU_SCALE = 1.0507009873554805
_SELU_ALPHA = 1.6732632423543772


def _selu(x):
    return _SELU_SCALE * jnp.where(x > 0, x, _SELU_ALPHA * (jnp.exp(x) - 1.0))


def _zero_rows(rows, nrow, w):
    def zr(i, _):
        def zc(c, _):
            rows[i, pl.ds(c * 16, 16)] = jnp.zeros((16,), jnp.float32)
            return 0
        return lax.fori_loop(0, w // 16, zc, 0)
    lax.fori_loop(0, nrow, zr, 0)


def _stage_chunk_and_logits(srcp, dstp, a_s_h, a_d_h, a_s, a_d,
                            src_l, dst_l, ee_l, wid):
    pltpu.sync_copy(a_s_h, a_s)
    pltpu.sync_copy(a_d_h, a_d)
    pltpu.sync_copy(srcp.at[wid], src_l)
    pltpu.sync_copy(dstp.at[wid], dst_l)

    def logits(j, _):
        def inner(kk, _):
            sl = pl.ds(kk * 16, 16)
            av = plsc.load_gather(a_s, [src_l[j, sl]])
            bv = plsc.load_gather(a_d, [dst_l[j, sl]])
            e = av + bv
            e = jnp.where(e > 0, e, 0.2 * e)
            ee_l[j, sl] = jnp.exp(e)
            return 0
        return lax.fori_loop(0, B // 16, inner, 0)
    lax.fori_loop(0, NB, logits, 0)


def _spmm_pass(table_h, acc, out_slice, src_l, dst_l, ee_l, rows, sem, sem2,
               Wc, sid):
    _zero_rows(rows.at[0], B, Wc)
    for q in range(SLICE // B):
        pltpu.sync_copy(rows.at[0], acc.at[pl.ds(sid * SLICE + q * B, B)])
    plsc.subcore_barrier()

    def spmm(g, _):
        descs = [pltpu.async_copy(
            table_h.at[src_l.at[g * GK + k]], rows.at[k], sem)
            for k in range(GK)]
        for d in descs:
            d.wait()
        for k in range(GK):
            j = g * GK + k

            def scale(kk, _, k=k, j=j):
                a16 = ee_l[j, pl.ds(kk * 16, 16)]
                for l in range(16):
                    a = a16[l]
                    r = kk * 16 + l
                    for c in range(Wc // 16):
                        sl = pl.ds(c * 16, 16)
                        rows[k, r, sl] = rows[k, r, sl] * a
                return 0
            lax.fori_loop(0, B // 16, scale, 0)
        descs = [pltpu.async_copy(
            rows.at[k], acc.at[dst_l.at[g * GK + k]], sem2, add=True)
            for k in range(GK)]
        for d in descs:
            d.wait()
        return 0
    lax.fori_loop(0, NB // GK, spmm, 0)

    plsc.subcore_barrier()
    sl = pl.ds(sid * SLICE, SLICE)
    pltpu.sync_copy(acc.at[sl], out_slice.at[sl])


def _gat1_body(srcp, dstp, a_s_h, a_d_h, t0, t1, t2, t3,
               out_h, dn16_h,
               a_s, a_d, src_l, dst_l, ee_l, rows, rows16, degrows,
               acc, acc16, sem, sem2):
    cid = lax.axis_index("c")
    sid = lax.axis_index("s")
    wid = sid * 2 + cid

    _stage_chunk_and_logits(srcp, dstp, a_s_h, a_d_h, a_s, a_d,
                            src_l, dst_l, ee_l, wid)

    _zero_rows(degrows, B, 16)
    iot = lax.iota(jnp.int32, 16)
    for kk in range(B // 16):
        plsc.store_scatter(degrows,
                           [kk * 16 + iot, jnp.full((16,), 1, jnp.int32)],
                           jnp.ones((16,), jnp.float32))
    _zero_rows(rows16.at[0], B, 16)
    for q in range(SLICE // B):
        pltpu.sync_copy(rows16.at[0],
                        acc16.at[pl.ds(sid * SLICE + q * B, B)])

    for half, table_h in enumerate((t0, t1, t2, t3)):
        _spmm_pass(table_h, acc, out_h.at[cid, half], src_l, dst_l, ee_l,
                   rows, sem, sem2, 32, sid)

    def dnpass(g, _):
        for k in range(GK):
            j = g * GK + k

            def fill(kk, _, k=k, j=j):
                ee16 = ee_l[j, pl.ds(kk * 16, 16)]
                plsc.store_scatter(
                    rows16, [jnp.full((16,), k, jnp.int32),
                             kk * 16 + iot,
                             jnp.zeros((16,), jnp.int32)], ee16)
                return 0
            lax.fori_loop(0, B // 16, fill, 0)
        descs = []
        for k in range(GK):
            j = g * GK + k
            descs.append(pltpu.async_copy(
                rows16.at[k], acc16.at[dst_l.at[j]], sem2, add=True))
            descs.append(pltpu.async_copy(
                degrows, acc16.at[src_l.at[j]], sem, add=True))
        for d in descs:
            d.wait()
        return 0
    lax.fori_loop(0, NB // GK, dnpass, 0)

    plsc.subcore_barrier()
    sl = pl.ds(sid * SLICE, SLICE)
    pltpu.sync_copy(acc16.at[sl], dn16_h.at[cid, sl])


def _sc_gat1():
    mesh = plsc.VectorSubcoreMesh(core_axis_name="c", subcore_axis_name="s")
    return pl.kernel(
        _gat1_body,
        out_type=(jax.ShapeDtypeStruct((2, 4, NP, 32), jnp.float32),
                  jax.ShapeDtypeStruct((2, NP, 16), jnp.float32)),
        mesh=mesh,
        scratch_types=(
            pltpu.VMEM((NP,), jnp.float32),
            pltpu.VMEM((NP,), jnp.float32),
            pltpu.VMEM((NB, B), jnp.int32),
            pltpu.VMEM((NB, B), jnp.int32),
            pltpu.VMEM((NB, B), jnp.float32),
            pltpu.VMEM((GK, B, 32), jnp.float32),
            pltpu.VMEM((GK, B, 16), jnp.float32),
            pltpu.VMEM((B, 16), jnp.float32),
            pltpu.VMEM_SHARED((NP, 32), jnp.float32),
            pltpu.VMEM_SHARED((NP, 16), jnp.float32),
            pltpu.SemaphoreType.DMA,
            pltpu.SemaphoreType.DMA,
        ),
        compiler_params=pltpu.CompilerParams(
            needs_layout_passes=False, use_tc_tiling_on_sc=False),
    )


def _gat32_body(srcp, dstp, a_s_h, a_d_h, table_h, out_h,
                a_s, a_d, src_l, dst_l, ee_l, rows, acc, sem, sem2):
    cid = lax.axis_index("c")
    sid = lax.axis_index("s")
    wid = sid * 2 + cid
    _stage_chunk_and_logits(srcp, dstp, a_s_h, a_d_h, a_s, a_d,
                            src_l, dst_l, ee_l, wid)
    _spmm_pass(table_h, acc, out_h.at[cid], src_l, dst_l, ee_l,
               rows, sem, sem2, 32, sid)


def _sc_gat32():
    mesh = plsc.VectorSubcoreMesh(core_axis_name="c", subcore_axis_name="s")
    return pl.kernel(
        _gat32_body,
        out_type=jax.ShapeDtypeStruct((2, NP, 32), jnp.float32),
        mesh=mesh,
        scratch_types=(
            pltpu.VMEM((NP,), jnp.float32),
            pltpu.VMEM((NP,), jnp.float32),
            pltpu.VMEM((NB, B), jnp.int32),
            pltpu.VMEM((NB, B), jnp.int32),
            pltpu.VMEM((NB, B), jnp.float32),
            pltpu.VMEM((GK, B, 32), jnp.float32),
            pltpu.VMEM_SHARED((NP, 32), jnp.float32),
            pltpu.SemaphoreType.DMA,
            pltpu.SemaphoreType.DMA,
        ),
        compiler_params=pltpu.CompilerParams(
            needs_layout_passes=False, use_tc_tiling_on_sc=False),
    )


def _smooth_body(Wd, GKs, colp, rowp, ta_h, tb_h, t_h, out_h,
                 ta, tb, col_l, row_l, w_l, rows, acc, sem, sem2):
    cid = lax.axis_index("c")
    sid = lax.axis_index("s")
    wid = sid * 2 + cid

    pltpu.sync_copy(ta_h, ta)
    pltpu.sync_copy(tb_h, tb)
    pltpu.sync_copy(colp.at[wid], col_l)
    pltpu.sync_copy(rowp.at[wid], row_l)

    def weights(j, _):
        def inner(kk, _):
            sl = pl.ds(kk * 16, 16)
            tbv = plsc.load_gather(tb, [col_l[j, sl]])
            w_l[j, sl] = plsc.load_gather(ta, [row_l[j, sl]]) * tbv
            return 0
        return lax.fori_loop(0, B // 16, inner, 0)
    lax.fori_loop(0, NB, weights, 0)

    _zero_rows(rows.at[0], B, Wd)
    for q in range(SLICE // B):
        pltpu.sync_copy(rows.at[0], acc.at[pl.ds(sid * SLICE + q * B, B)])
    plsc.subcore_barrier()

    def spmm(g, _):
        descs = [pltpu.async_copy(
            t_h.at[col_l.at[g * GKs + k]], rows.at[k], sem)
            for k in range(GKs)]
        for d in descs:
            d.wait()
        for k in range(GKs):
            j = g * GKs + k

            def scale(kk, _, k=k, j=j):
                a16 = w_l[j, pl.ds(kk * 16, 16)]
                for l in range(16):
                    a = a16[l]
                    r = kk * 16 + l
                    for c in range(Wd // 16):
                        sl = pl.ds(c * 16, 16)
                        rows[k, r, sl] = rows[k, r, sl] * a
                return 0
            lax.fori_loop(0, B // 16, scale, 0)
        descs = [pltpu.async_copy(
            rows.at[k], acc.at[row_l.at[g * GKs + k]], sem2, add=True)
            for k in range(GKs)]
        for d in descs:
            d.wait()
        return 0
    lax.fori_loop(0, NB // GKs, spmm, 0)

    plsc.subcore_barrier()
    sl = pl.ds(sid * SLICE, SLICE)
    pltpu.sync_copy(acc.at[sl], out_h.at[cid, sl])


def _sc_smooth(Wd):
    GKs = 2 if Wd == 64 else 4
    mesh = plsc.VectorSubcoreMesh(core_axis_name="c", subcore_axis_name="s")
    return pl.kernel(
        functools.partial(_smooth_body, Wd, GKs),
        out_type=jax.ShapeDtypeStruct((2, NP, Wd), jnp.float32),
        mesh=mesh,
        scratch_types=(
            pltpu.VMEM((NP,), jnp.float32),
            pltpu.VMEM((NP,), jnp.float32),
            pltpu.VMEM((NB, B), jnp.int32),
            pltpu.VMEM((NB, B), jnp.int32),
            pltpu.VMEM((NB, B), jnp.float32),
            pltpu.VMEM((GKs, B, Wd), jnp.float32),
            pltpu.VMEM_SHARED((NP, Wd), jnp.float32),
            pltpu.SemaphoreType.DMA,
            pltpu.SemaphoreType.DMA,
        ),
        compiler_params=pltpu.CompilerParams(
            needs_layout_passes=False, use_tc_tiling_on_sc=False),
    )


_BLK = 1024
_G = NP // _BLK


def _tc0_body(x_ref, v1_ref, o1_ref, o2_ref):
    x = x_ref[...]
    o1_ref[...] = jnp.sum(x * v1_ref[0][None, :], axis=1)
    o2_ref[...] = jnp.sum(x * v1_ref[1][None, :], axis=1)


def _tc0(x_pad, v1s, v1d):
    return pl.pallas_call(
        _tc0_body,
        out_shape=(jax.ShapeDtypeStruct((NP,), jnp.float32),
                   jax.ShapeDtypeStruct((NP,), jnp.float32)),
        grid=(_G,),
        in_specs=[pl.BlockSpec((_BLK, 128), lambda i: (i, 0)),
                  pl.BlockSpec((2, 128), lambda i: (0, 0))],
        out_specs=(pl.BlockSpec((_BLK,), lambda i: (i,)),
                   pl.BlockSpec((_BLK,), lambda i: (i,))),
    )(x_pad, jnp.stack([v1s, v1d]))


def _tc1_body(p_ref, d_ref, w1_ref, w2p_ref, v2_ref,
              xs2_ref, a2s_ref, a2d_ref, deg_ref):
    m = jnp.concatenate([p_ref[0, q] + p_ref[1, q] for q in range(4)],
                        axis=1)
    dn = d_ref[0, :, 0] + d_ref[1, :, 0]
    m = m * (1.0 / (dn + 1e-16))[:, None]
    h1 = _selu(jnp.dot(m, w1_ref[...], preferred_element_type=jnp.float32))
    xs2 = jnp.dot(h1, w2p_ref[...], preferred_element_type=jnp.float32)
    lane = lax.broadcasted_iota(jnp.int32, (_BLK, 32), 1)
    xs2_ref[...] = jnp.where(lane == 31, 1.0, xs2)
    a2s_ref[...] = jnp.sum(h1 * v2_ref[0][None, :], axis=1)
    a2d_ref[...] = jnp.sum(h1 * v2_ref[1][None, :], axis=1)
    deg_ref[...] = d_ref[0, :, 1] + d_ref[1, :, 1]


def _tc1(m1P, dn16P, W1_src, W2p, v2s, v2d):
    return pl.pallas_call(
        _tc1_body,
        out_shape=(jax.ShapeDtypeStruct((NP, 32), jnp.float32),
                   jax.ShapeDtypeStruct((NP,), jnp.float32),
                   jax.ShapeDtypeStruct((NP,), jnp.float32),
                   jax.ShapeDtypeStruct((NP,), jnp.float32)),
        grid=(_G,),
        in_specs=[pl.BlockSpec((2, 4, _BLK, 32), lambda i: (0, 0, i, 0)),
                  pl.BlockSpec((2, _BLK, 16), lambda i: (0, i, 0)),
                  pl.BlockSpec((128, 512), lambda i: (0, 0)),
                  pl.BlockSpec((512, 32), lambda i: (0, 0)),
                  pl.BlockSpec((2, 512), lambda i: (0, 0))],
        out_specs=(pl.BlockSpec((_BLK, 32), lambda i: (i, 0)),
                   pl.BlockSpec((_BLK,), lambda i: (i,)),
                   pl.BlockSpec((_BLK,), lambda i: (i,)),
                   pl.BlockSpec((_BLK,), lambda i: (i,))),
    )(m1P, dn16P, W1_src, W2p, jnp.stack([v2s, v2d]))


def _tc2_body(p_ref, deg_ref, v3_ref, sa_ref, h2t_ref, h2ab_ref,
              a3s_ref, a3d_ref, dis_ref, dis2ad_ref, dis2_ref):
    raw = p_ref[0] + p_ref[1]
    dn = raw[:, 31]
    h2 = raw * (1.0 / (dn + 1e-16))[:, None]
    lane = lax.broadcasted_iota(jnp.int32, (_BLK, 32), 1)
    h2t_ref[...] = jnp.where(lane == 31, 1.0, h2)
    h2ab_ref[...] = jnp.concatenate(
        [h2 * sa_ref[0, 0], h2 * (2.0 * sa_ref[0, 1])], axis=1)
    a3s_ref[...] = jnp.sum(h2 * v3_ref[0][None, :], axis=1)
    a3d_ref[...] = jnp.sum(h2 * v3_ref[1][None, :], axis=1)
    deg = deg_ref[...]
    logd = jnp.log(jnp.maximum(deg, 1e-30))
    dis_ref[...] = jnp.where(deg > 0, jnp.exp(-0.6 * logd), 0.0)
    dis2 = jnp.where(deg > 0, jnp.exp(-0.5 * logd), 0.0)
    dis2_ref[...] = dis2
    alpha_d = 1.0 / (1.0 + jnp.log(deg + 1.0))
    dis2ad_ref[...] = dis2 * alpha_d


def _tc2(h2P, deg, v3s, v3d, sa1, sa2):
    return pl.pallas_call(
        _tc2_body,
        out_shape=(jax.ShapeDtypeStruct((NP, 32), jnp.float32),
                   jax.ShapeDtypeStruct((NP, 64), jnp.float32),
                   jax.ShapeDtypeStruct((NP,), jnp.float32),
                   jax.ShapeDtypeStruct((NP,), jnp.float32),
                   jax.ShapeDtypeStruct((NP,), jnp.float32),
                   jax.ShapeDtypeStruct((NP,), jnp.float32),
                   jax.ShapeDtypeStruct((NP,), jnp.float32)),
        grid=(_G,),
        in_specs=[pl.BlockSpec((2, _BLK, 32), lambda i: (0, i, 0)),
                  pl.BlockSpec((_BLK,), lambda i: (i,)),
                  pl.BlockSpec((2, 32), lambda i: (0, 0)),
                  pl.BlockSpec(memory_space=pltpu.SMEM)],
        out_specs=(pl.BlockSpec((_BLK, 32), lambda i: (i, 0)),
                   pl.BlockSpec((_BLK, 64), lambda i: (i, 0)),
                   pl.BlockSpec((_BLK,), lambda i: (i,)),
                   pl.BlockSpec((_BLK,), lambda i: (i,)),
                   pl.BlockSpec((_BLK,), lambda i: (i,)),
                   pl.BlockSpec((_BLK,), lambda i: (i,)),
                   pl.BlockSpec((_BLK,), lambda i: (i,))),
    )(h2P, deg, jnp.stack([v3s, v3d]),
      jnp.stack([sa1, sa2]).reshape(1, 2))


def _tc3_body(p_ref, w2t_ref, w1t_ref, h4_ref):
    raw = p_ref[0] + p_ref[1]
    dn = raw[:, 31]
    m3 = raw * (1.0 / (dn + 1e-16))[:, None]
    h3 = _selu(jnp.dot(m3, w2t_ref[...], preferred_element_type=jnp.float32))
    h4_ref[...] = jnp.dot(h3, w1t_ref[...], preferred_element_type=jnp.float32)


def _tc3(m3P, W2pT, W1T):
    return pl.pallas_call(
        _tc3_body,
        out_shape=jax.ShapeDtypeStruct((NP, 128), jnp.float32),
        grid=(_G,),
        in_specs=[pl.BlockSpec((2, _BLK, 32), lambda i: (0, i, 0)),
                  pl.BlockSpec((32, 512), lambda i: (0, 0)),
                  pl.BlockSpec((512, 128), lambda i: (0, 0))],
        out_specs=pl.BlockSpec((_BLK, 128), lambda i: (i, 0)),
    )(m3P, W2pT, W1T)


def _combine_body(a_ref, o_ref, *, k, scale):
    acc = a_ref[0]
    for i in range(1, k):
        acc = acc + a_ref[i]
    o_ref[...] = acc * scale


def _combine(parts, scale=1.0):
    k, np_, w = parts.shape
    return pl.pallas_call(
        functools.partial(_combine_body, k=k, scale=scale),
        out_shape=jax.ShapeDtypeStruct((np_, w), jnp.float32),
        grid=(np_ // _BLK,),
        in_specs=[pl.BlockSpec((k, _BLK, w), lambda i: (0, i, 0))],
        out_specs=pl.BlockSpec((_BLK, w), lambda i: (i, 0)),
    )(parts)


def _combine2s_body(p_ref, s_ref, o_ref):
    s = p_ref[0] + p_ref[1]
    lane = lax.broadcasted_iota(jnp.int32, (_BLK, 64), 1)
    fac = jnp.where(lane < 32, s_ref[0, 0], 2.0 * s_ref[0, 1])
    o_ref[...] = s * fac


def _combine2s(p, sa1, sa2):
    return pl.pallas_call(
        _combine2s_body,
        out_shape=jax.ShapeDtypeStruct((NP, 64), jnp.float32),
        grid=(_G,),
        in_specs=[pl.BlockSpec((2, _BLK, 64), lambda i: (0, i, 0)),
                  pl.BlockSpec(memory_space=pltpu.SMEM)],
        out_specs=pl.BlockSpec((_BLK, 64), lambda i: (i, 0)),
    )(p, jnp.stack([sa1, sa2]).reshape(1, 2))


def _combine_mid_body(p_ref, o_ref):
    s = p_ref[0] + p_ref[1]
    o_ref[...] = 0.5 * (s[:, :32] + s[:, 32:])


def _combine_mid(p):
    return pl.pallas_call(
        _combine_mid_body,
        out_shape=jax.ShapeDtypeStruct((NP, 32), jnp.float32),
        grid=(_G,),
        in_specs=[pl.BlockSpec((2, _BLK, 64), lambda i: (0, i, 0))],
        out_specs=pl.BlockSpec((_BLK, 32), lambda i: (i, 0)),
    )(p)


def kernel(features, edge_index, W1_src, W1_dst, att1_src, att1_dst,
           W2_src, W2_dst, att2_src, att2_dst, att3_src, att3_dst,
           smooth_alpha_1, smooth_alpha_2):
    src = edge_index[0]
    dst = edge_index[1]
    pad_e = EPAD - E
    srcp = jnp.concatenate([src, jnp.full((pad_e,), N, jnp.int32)]
                           ).reshape(32, NB, B)
    dstp = jnp.concatenate([dst, jnp.full((pad_e,), N, jnp.int32)]
                           ).reshape(32, NB, B)

    x_pad = jnp.pad(features, ((0, NP - N), (0, 0)))
    W2p = jnp.pad(W2_src, ((0, 0), (0, 2)))
    W2pT = jnp.pad(W2_src.T, ((0, 2), (0, 0)))
    W1T = W1_src.T
    v1s = W1_src @ att1_src
    v1d = W1_dst @ att1_dst
    v2s = W2_src @ att2_src
    v2d = W2_dst @ att2_dst
    v3s = jnp.pad(W2_src.T @ att3_src, (0, 2))
    v3d = jnp.pad(W2_dst.T @ att3_dst, (0, 2))

    a1s, a1d = _tc0(x_pad, v1s, v1d)
    m1P, dn16P = _sc_gat1()(srcp, dstp, a1s, a1d,
                            x_pad[:, :32], x_pad[:, 32:64],
                            x_pad[:, 64:96], x_pad[:, 96:])
    xs2, a2s, a2d, deg = _tc1(m1P, dn16P, W1_src, W2p, v2s, v2d)

    h2P = _sc_gat32()(srcp, dstp, a2s, a2d, xs2)
    h2t, h2ab, a3s, a3d, dis, dis2ad, dis2 = _tc2(
        h2P, deg, v3s, v3d, smooth_alpha_1, smooth_alpha_2)
    m3P = _sc_gat32()(srcp, dstp, a3s, a3d, h2t)
    h4 = _tc3(m3P, W2pT, W1T)

    p = _sc_smooth(64)(dstp, srcp, dis, dis, h2ab)
    t2 = _combine2s(p, smooth_alpha_1, smooth_alpha_2)
    p = _sc_smooth(64)(dstp, srcp, dis, dis, t2)
    sm = _combine_mid(p)
    smP = _sc_smooth(32)(dstp, srcp, dis2ad, dis2, sm)
    sm = _combine(smP)
    smP = _sc_smooth(32)(dstp, srcp, dis2ad, dis2, sm)
    sm = _combine(smP)

    return (sm[:N, :30], h4[:N])


# speedup vs baseline: 1.0583x; 1.0442x over previous
import functools

import jax
import jax.numpy as jnp
from jax import lax
from jax.experimental import pallas as pl
from jax.experimental.pallas import tpu as pltpu
from jax.experimental.pallas import tpu_sc as plsc

N = 10000
E = 320000
NP = 10240
B = 128
NB = 80
EPT = NB * B
EPAD = 32 * EPT
SLICE = NP // 16
GK = 2

_SELU_SCALE = 1.0507009873554805
_SELU_ALPHA = 1.6732632423543772


def _selu(x):
    return _SELU_SCALE * jnp.where(x > 0, x, _SELU_ALPHA * (jnp.exp(x) - 1.0))


def _zero_rows(rows, nrow, w):
    def zr(i, _):
        def zc(c, _):
            rows[i, pl.ds(c * 16, 16)] = jnp.zeros((16,), jnp.float32)
            return 0
        return lax.fori_loop(0, w // 16, zc, 0)
    lax.fori_loop(0, nrow, zr, 0)


def _stage_chunk_and_logits(srcp, dstp, a_s_h, a_d_h, a_s, a_d,
                            src_l, dst_l, ee_l, wid):
    pltpu.sync_copy(a_s_h, a_s)
    pltpu.sync_copy(a_d_h, a_d)
    pltpu.sync_copy(srcp.at[wid], src_l)
    pltpu.sync_copy(dstp.at[wid], dst_l)

    def logits(j, _):
        def inner(kk, _):
            sl = pl.ds(kk * 16, 16)
            av = plsc.load_gather(a_s, [src_l[j, sl]])
            bv = plsc.load_gather(a_d, [dst_l[j, sl]])
            e = av + bv
            e = jnp.where(e > 0, e, 0.2 * e)
            ee_l[j, sl] = jnp.exp(e)
            return 0
        return lax.fori_loop(0, B // 16, inner, 0)
    lax.fori_loop(0, NB, logits, 0)


def _spmm_pass(table_h, acc, out_slice, src_l, dst_l, ee_l, rows, sem, sem2,
               Wc, sid):
    _zero_rows(rows.at[0], B, Wc)
    for q in range(SLICE // B):
        pltpu.sync_copy(rows.at[0], acc.at[pl.ds(sid * SLICE + q * B, B)])
    plsc.subcore_barrier()

    def spmm(g, _):
        descs = [pltpu.async_copy(
            table_h.at[src_l.at[g * GK + k]], rows.at[k], sem)
            for k in range(GK)]
        for d in descs:
            d.wait()
        for k in range(GK):
            j = g * GK + k

            def scale(kk, _, k=k, j=j):
                a16 = ee_l[j, pl.ds(kk * 16, 16)]
                for l in range(16):
                    a = a16[l]
                    r = kk * 16 + l
                    for c in range(Wc // 16):
                        sl = pl.ds(c * 16, 16)
                        rows[k, r, sl] = rows[k, r, sl] * a
                return 0
            lax.fori_loop(0, B // 16, scale, 0)
        descs = [pltpu.async_copy(
            rows.at[k], acc.at[dst_l.at[g * GK + k]], sem2, add=True)
            for k in range(GK)]
        for d in descs:
            d.wait()
        return 0
    lax.fori_loop(0, NB // GK, spmm, 0)

    plsc.subcore_barrier()
    sl = pl.ds(sid * SLICE, SLICE)
    pltpu.sync_copy(acc.at[sl], out_slice.at[sl])


def _gat1_body(srcp, dstp, a_s_h, a_d_h, t0, t1, t2, t3,
               out_h, dn16_h,
               a_s, a_d, src_l, dst_l, ee_l, rows, rows16, degrows,
               acc, acc16, sem, sem2):
    cid = lax.axis_index("c")
    sid = lax.axis_index("s")
    wid = sid * 2 + cid

    _stage_chunk_and_logits(srcp, dstp, a_s_h, a_d_h, a_s, a_d,
                            src_l, dst_l, ee_l, wid)

    _zero_rows(degrows, B, 16)
    iot = lax.iota(jnp.int32, 16)
    for kk in range(B // 16):
        plsc.store_scatter(degrows,
                           [kk * 16 + iot, jnp.full((16,), 1, jnp.int32)],
                           jnp.ones((16,), jnp.float32))
    _zero_rows(rows16.at[0], B, 16)
    for q in range(SLICE // B):
        pltpu.sync_copy(rows16.at[0],
                        acc16.at[pl.ds(sid * SLICE + q * B, B)])

    for half, table_h in enumerate((t0, t1, t2, t3)):
        _spmm_pass(table_h, acc, out_h.at[cid, half], src_l, dst_l, ee_l,
                   rows, sem, sem2, 32, sid)

    def dnpass(g, _):
        for k in range(GK):
            j = g * GK + k

            def fill(kk, _, k=k, j=j):
                ee16 = ee_l[j, pl.ds(kk * 16, 16)]
                plsc.store_scatter(
                    rows16, [jnp.full((16,), k, jnp.int32),
                             kk * 16 + iot,
                             jnp.zeros((16,), jnp.int32)], ee16)
                return 0
            lax.fori_loop(0, B // 16, fill, 0)
        descs = []
        for k in range(GK):
            j = g * GK + k
            descs.append(pltpu.async_copy(
                rows16.at[k], acc16.at[dst_l.at[j]], sem2, add=True))
            descs.append(pltpu.async_copy(
                degrows, acc16.at[src_l.at[j]], sem, add=True))
        for d in descs:
            d.wait()
        return 0
    lax.fori_loop(0, NB // GK, dnpass, 0)

    plsc.subcore_barrier()
    sl = pl.ds(sid * SLICE, SLICE)
    pltpu.sync_copy(acc16.at[sl], dn16_h.at[cid, sl])


def _sc_gat1():
    mesh = plsc.VectorSubcoreMesh(core_axis_name="c", subcore_axis_name="s")
    return pl.kernel(
        _gat1_body,
        out_type=(jax.ShapeDtypeStruct((2, 4, NP, 32), jnp.float32),
                  jax.ShapeDtypeStruct((2, NP, 16), jnp.float32)),
        mesh=mesh,
        scratch_types=(
            pltpu.VMEM((NP,), jnp.float32),
            pltpu.VMEM((NP,), jnp.float32),
            pltpu.VMEM((NB, B), jnp.int32),
            pltpu.VMEM((NB, B), jnp.int32),
            pltpu.VMEM((NB, B), jnp.float32),
            pltpu.VMEM((GK, B, 32), jnp.float32),
            pltpu.VMEM((GK, B, 16), jnp.float32),
            pltpu.VMEM((B, 16), jnp.float32),
            pltpu.VMEM_SHARED((NP, 32), jnp.float32),
            pltpu.VMEM_SHARED((NP, 16), jnp.float32),
            pltpu.SemaphoreType.DMA,
            pltpu.SemaphoreType.DMA,
        ),
        compiler_params=pltpu.CompilerParams(
            needs_layout_passes=False, use_tc_tiling_on_sc=False),
    )


def _gat32_body(srcp, dstp, a_s_h, a_d_h, table_h, out_h,
                a_s, a_d, src_l, dst_l, ee_l, rows, acc, sem, sem2):
    cid = lax.axis_index("c")
    sid = lax.axis_index("s")
    wid = sid * 2 + cid
    _stage_chunk_and_logits(srcp, dstp, a_s_h, a_d_h, a_s, a_d,
                            src_l, dst_l, ee_l, wid)
    _spmm_pass(table_h, acc, out_h.at[cid], src_l, dst_l, ee_l,
               rows, sem, sem2, 32, sid)


def _sc_gat32():
    mesh = plsc.VectorSubcoreMesh(core_axis_name="c", subcore_axis_name="s")
    return pl.kernel(
        _gat32_body,
        out_type=jax.ShapeDtypeStruct((2, NP, 32), jnp.float32),
        mesh=mesh,
        scratch_types=(
            pltpu.VMEM((NP,), jnp.float32),
            pltpu.VMEM((NP,), jnp.float32),
            pltpu.VMEM((NB, B), jnp.int32),
            pltpu.VMEM((NB, B), jnp.int32),
            pltpu.VMEM((NB, B), jnp.float32),
            pltpu.VMEM((GK, B, 32), jnp.float32),
            pltpu.VMEM_SHARED((NP, 32), jnp.float32),
            pltpu.SemaphoreType.DMA,
            pltpu.SemaphoreType.DMA,
        ),
        compiler_params=pltpu.CompilerParams(
            needs_layout_passes=False, use_tc_tiling_on_sc=False),
    )


def _smooth_body(Wd, GKs, colp, rowp, ta_h, tb_h, t_h, out_h,
                 ta, tb, col_l, row_l, w_l, rows, acc, sem, sem2):
    cid = lax.axis_index("c")
    sid = lax.axis_index("s")
    wid = sid * 2 + cid

    pltpu.sync_copy(ta_h, ta)
    pltpu.sync_copy(tb_h, tb)
    pltpu.sync_copy(colp.at[wid], col_l)
    pltpu.sync_copy(rowp.at[wid], row_l)

    def weights(j, _):
        def inner(kk, _):
            sl = pl.ds(kk * 16, 16)
            tbv = plsc.load_gather(tb, [col_l[j, sl]])
            w_l[j, sl] = plsc.load_gather(ta, [row_l[j, sl]]) * tbv
            return 0
        return lax.fori_loop(0, B // 16, inner, 0)
    lax.fori_loop(0, NB, weights, 0)

    _zero_rows(rows.at[0], B, Wd)
    for q in range(SLICE // B):
        pltpu.sync_copy(rows.at[0], acc.at[pl.ds(sid * SLICE + q * B, B)])
    plsc.subcore_barrier()

    def spmm(g, _):
        descs = [pltpu.async_copy(
            t_h.at[col_l.at[g * GKs + k]], rows.at[k], sem)
            for k in range(GKs)]
        for d in descs:
            d.wait()
        for k in range(GKs):
            j = g * GKs + k

            def scale(kk, _, k=k, j=j):
                a16 = w_l[j, pl.ds(kk * 16, 16)]
                for l in range(16):
                    a = a16[l]
                    r = kk * 16 + l
                    for c in range(Wd // 16):
                        sl = pl.ds(c * 16, 16)
                        rows[k, r, sl] = rows[k, r, sl] * a
                return 0
            lax.fori_loop(0, B // 16, scale, 0)
        descs = [pltpu.async_copy(
            rows.at[k], acc.at[row_l.at[g * GKs + k]], sem2, add=True)
            for k in range(GKs)]
        for d in descs:
            d.wait()
        return 0
    lax.fori_loop(0, NB // GKs, spmm, 0)

    plsc.subcore_barrier()
    sl = pl.ds(sid * SLICE, SLICE)
    pltpu.sync_copy(acc.at[sl], out_h.at[cid, sl])


def _sc_smooth(Wd):
    GKs = 4
    mesh = plsc.VectorSubcoreMesh(core_axis_name="c", subcore_axis_name="s")
    return pl.kernel(
        functools.partial(_smooth_body, Wd, GKs),
        out_type=jax.ShapeDtypeStruct((2, NP, Wd), jnp.float32),
        mesh=mesh,
        scratch_types=(
            pltpu.VMEM((NP,), jnp.float32),
            pltpu.VMEM((NP,), jnp.float32),
            pltpu.VMEM((NB, B), jnp.int32),
            pltpu.VMEM((NB, B), jnp.int32),
            pltpu.VMEM((NB, B), jnp.float32),
            pltpu.VMEM((GKs, B, Wd), jnp.float32),
            pltpu.VMEM_SHARED((NP, Wd), jnp.float32),
            pltpu.SemaphoreType.DMA,
            pltpu.SemaphoreType.DMA,
        ),
        compiler_params=pltpu.CompilerParams(
            needs_layout_passes=False, use_tc_tiling_on_sc=False),
    )


_BLK = 1024
_G = NP // _BLK


def _tc0_body(x_ref, v1_ref, o1_ref, o2_ref):
    x = x_ref[...]
    o1_ref[...] = jnp.sum(x * v1_ref[0][None, :], axis=1)
    o2_ref[...] = jnp.sum(x * v1_ref[1][None, :], axis=1)


def _tc0(x_pad, v1s, v1d):
    return pl.pallas_call(
        _tc0_body,
        out_shape=(jax.ShapeDtypeStruct((NP,), jnp.float32),
                   jax.ShapeDtypeStruct((NP,), jnp.float32)),
        grid=(_G,),
        in_specs=[pl.BlockSpec((_BLK, 128), lambda i: (i, 0)),
                  pl.BlockSpec((2, 128), lambda i: (0, 0))],
        out_specs=(pl.BlockSpec((_BLK,), lambda i: (i,)),
                   pl.BlockSpec((_BLK,), lambda i: (i,))),
    )(x_pad, jnp.stack([v1s, v1d]))


def _tc1_body(p_ref, d_ref, w1_ref, w2p_ref, v2_ref,
              xs2_ref, a2s_ref, a2d_ref, deg_ref):
    m = jnp.concatenate([p_ref[0, q] + p_ref[1, q] for q in range(4)],
                        axis=1)
    dn = d_ref[0, :, 0] + d_ref[1, :, 0]
    m = m * (1.0 / (dn + 1e-16))[:, None]
    h1 = _selu(jnp.dot(m, w1_ref[...], preferred_element_type=jnp.float32))
    xs2 = jnp.dot(h1, w2p_ref[...], preferred_element_type=jnp.float32)
    lane = lax.broadcasted_iota(jnp.int32, (_BLK, 32), 1)
    xs2_ref[...] = jnp.where(lane == 31, 1.0, xs2)
    a2s_ref[...] = jnp.sum(h1 * v2_ref[0][None, :], axis=1)
    a2d_ref[...] = jnp.sum(h1 * v2_ref[1][None, :], axis=1)
    deg_ref[...] = d_ref[0, :, 1] + d_ref[1, :, 1]


def _tc1(m1P, dn16P, W1_src, W2p, v2s, v2d):
    return pl.pallas_call(
        _tc1_body,
        out_shape=(jax.ShapeDtypeStruct((NP, 32), jnp.float32),
                   jax.ShapeDtypeStruct((NP,), jnp.float32),
                   jax.ShapeDtypeStruct((NP,), jnp.float32),
                   jax.ShapeDtypeStruct((NP,), jnp.float32)),
        grid=(_G,),
        in_specs=[pl.BlockSpec((2, 4, _BLK, 32), lambda i: (0, 0, i, 0)),
                  pl.BlockSpec((2, _BLK, 16), lambda i: (0, i, 0)),
                  pl.BlockSpec((128, 512), lambda i: (0, 0)),
                  pl.BlockSpec((512, 32), lambda i: (0, 0)),
                  pl.BlockSpec((2, 512), lambda i: (0, 0))],
        out_specs=(pl.BlockSpec((_BLK, 32), lambda i: (i, 0)),
                   pl.BlockSpec((_BLK,), lambda i: (i,)),
                   pl.BlockSpec((_BLK,), lambda i: (i,)),
                   pl.BlockSpec((_BLK,), lambda i: (i,))),
    )(m1P, dn16P, W1_src, W2p, jnp.stack([v2s, v2d]))


def _tc2_body(p_ref, deg_ref, v3_ref, sa_ref, h2t_ref, h2ab_ref,
              a3s_ref, a3d_ref, dis_ref, dis2ad_ref, dis2_ref):
    raw = p_ref[0] + p_ref[1]
    dn = raw[:, 31]
    h2 = raw * (1.0 / (dn + 1e-16))[:, None]
    lane = lax.broadcasted_iota(jnp.int32, (_BLK, 32), 1)
    h2t_ref[...] = jnp.where(lane == 31, 1.0, h2)
    h2ab_ref[...] = jnp.concatenate(
        [h2 * sa_ref[0, 0], h2 * (2.0 * sa_ref[0, 1])], axis=1)
    a3s_ref[...] = jnp.sum(h2 * v3_ref[0][None, :], axis=1)
    a3d_ref[...] = jnp.sum(h2 * v3_ref[1][None, :], axis=1)
    deg = deg_ref[...]
    logd = jnp.log(jnp.maximum(deg, 1e-30))
    dis_ref[...] = jnp.where(deg > 0, jnp.exp(-0.6 * logd), 0.0)
    dis2 = jnp.where(deg > 0, jnp.exp(-0.5 * logd), 0.0)
    dis2_ref[...] = dis2
    alpha_d = 1.0 / (1.0 + jnp.log(deg + 1.0))
    dis2ad_ref[...] = dis2 * alpha_d


def _tc2(h2P, deg, v3s, v3d, sa1, sa2):
    return pl.pallas_call(
        _tc2_body,
        out_shape=(jax.ShapeDtypeStruct((NP, 32), jnp.float32),
                   jax.ShapeDtypeStruct((NP, 64), jnp.float32),
                   jax.ShapeDtypeStruct((NP,), jnp.float32),
                   jax.ShapeDtypeStruct((NP,), jnp.float32),
                   jax.ShapeDtypeStruct((NP,), jnp.float32),
                   jax.ShapeDtypeStruct((NP,), jnp.float32),
                   jax.ShapeDtypeStruct((NP,), jnp.float32)),
        grid=(_G,),
        in_specs=[pl.BlockSpec((2, _BLK, 32), lambda i: (0, i, 0)),
                  pl.BlockSpec((_BLK,), lambda i: (i,)),
                  pl.BlockSpec((2, 32), lambda i: (0, 0)),
                  pl.BlockSpec(memory_space=pltpu.SMEM)],
        out_specs=(pl.BlockSpec((_BLK, 32), lambda i: (i, 0)),
                   pl.BlockSpec((_BLK, 64), lambda i: (i, 0)),
                   pl.BlockSpec((_BLK,), lambda i: (i,)),
                   pl.BlockSpec((_BLK,), lambda i: (i,)),
                   pl.BlockSpec((_BLK,), lambda i: (i,)),
                   pl.BlockSpec((_BLK,), lambda i: (i,)),
                   pl.BlockSpec((_BLK,), lambda i: (i,))),
    )(h2P, deg, jnp.stack([v3s, v3d]),
      jnp.stack([sa1, sa2]).reshape(1, 2))


def _tc3_body(p_ref, w2t_ref, w1t_ref, h4_ref):
    raw = p_ref[0] + p_ref[1]
    dn = raw[:, 31]
    m3 = raw * (1.0 / (dn + 1e-16))[:, None]
    h3 = _selu(jnp.dot(m3, w2t_ref[...], preferred_element_type=jnp.float32))
    h4_ref[...] = jnp.dot(h3, w1t_ref[...], preferred_element_type=jnp.float32)


def _tc3(m3P, W2pT, W1T):
    return pl.pallas_call(
        _tc3_body,
        out_shape=jax.ShapeDtypeStruct((NP, 128), jnp.float32),
        grid=(_G,),
        in_specs=[pl.BlockSpec((2, _BLK, 32), lambda i: (0, i, 0)),
                  pl.BlockSpec((32, 512), lambda i: (0, 0)),
                  pl.BlockSpec((512, 128), lambda i: (0, 0))],
        out_specs=pl.BlockSpec((_BLK, 128), lambda i: (i, 0)),
    )(m3P, W2pT, W1T)


def _combine_body(a_ref, o_ref, *, k, scale):
    acc = a_ref[0]
    for i in range(1, k):
        acc = acc + a_ref[i]
    o_ref[...] = acc * scale


def _combine(parts, scale=1.0):
    k, np_, w = parts.shape
    return pl.pallas_call(
        functools.partial(_combine_body, k=k, scale=scale),
        out_shape=jax.ShapeDtypeStruct((np_, w), jnp.float32),
        grid=(np_ // _BLK,),
        in_specs=[pl.BlockSpec((k, _BLK, w), lambda i: (0, i, 0))],
        out_specs=pl.BlockSpec((_BLK, w), lambda i: (i, 0)),
    )(parts)


def _combine2s_body(p_ref, s_ref, o_ref):
    s = p_ref[0] + p_ref[1]
    lane = lax.broadcasted_iota(jnp.int32, (_BLK, 64), 1)
    fac = jnp.where(lane < 32, s_ref[0, 0], 2.0 * s_ref[0, 1])
    o_ref[...] = s * fac


def _combine2s(p, sa1, sa2):
    return pl.pallas_call(
        _combine2s_body,
        out_shape=jax.ShapeDtypeStruct((NP, 64), jnp.float32),
        grid=(_G,),
        in_specs=[pl.BlockSpec((2, _BLK, 64), lambda i: (0, i, 0)),
                  pl.BlockSpec(memory_space=pltpu.SMEM)],
        out_specs=pl.BlockSpec((_BLK, 64), lambda i: (i, 0)),
    )(p, jnp.stack([sa1, sa2]).reshape(1, 2))


def _combine_mid_body(p_ref, o_ref):
    s = p_ref[0] + p_ref[1]
    o_ref[...] = 0.5 * (s[:, :32] + s[:, 32:])


def _combine_mid(p):
    return pl.pallas_call(
        _combine_mid_body,
        out_shape=jax.ShapeDtypeStruct((NP, 32), jnp.float32),
        grid=(_G,),
        in_specs=[pl.BlockSpec((2, _BLK, 64), lambda i: (0, i, 0))],
        out_specs=pl.BlockSpec((_BLK, 32), lambda i: (i, 0)),
    )(p)


def kernel(features, edge_index, W1_src, W1_dst, att1_src, att1_dst,
           W2_src, W2_dst, att2_src, att2_dst, att3_src, att3_dst,
           smooth_alpha_1, smooth_alpha_2):
    src = edge_index[0]
    dst = edge_index[1]
    pad_e = EPAD - E
    srcp = jnp.concatenate([src, jnp.full((pad_e,), N, jnp.int32)]
                           ).reshape(32, NB, B)
    dstp = jnp.concatenate([dst, jnp.full((pad_e,), N, jnp.int32)]
                           ).reshape(32, NB, B)

    x_pad = jnp.pad(features, ((0, NP - N), (0, 0)))
    W2p = jnp.pad(W2_src, ((0, 0), (0, 2)))
    W2pT = jnp.pad(W2_src.T, ((0, 2), (0, 0)))
    W1T = W1_src.T
    v1s = W1_src @ att1_src
    v1d = W1_dst @ att1_dst
    v2s = W2_src @ att2_src
    v2d = W2_dst @ att2_dst
    v3s = jnp.pad(W2_src.T @ att3_src, (0, 2))
    v3d = jnp.pad(W2_dst.T @ att3_dst, (0, 2))

    a1s, a1d = _tc0(x_pad, v1s, v1d)
    m1P, dn16P = _sc_gat1()(srcp, dstp, a1s, a1d,
                            x_pad[:, :32], x_pad[:, 32:64],
                            x_pad[:, 64:96], x_pad[:, 96:])
    xs2, a2s, a2d, deg = _tc1(m1P, dn16P, W1_src, W2p, v2s, v2d)

    h2P = _sc_gat32()(srcp, dstp, a2s, a2d, xs2)
    h2t, h2ab, a3s, a3d, dis, dis2ad, dis2 = _tc2(
        h2P, deg, v3s, v3d, smooth_alpha_1, smooth_alpha_2)
    m3P = _sc_gat32()(srcp, dstp, a3s, a3d, h2t)
    h4 = _tc3(m3P, W2pT, W1T)

    p = _sc_smooth(64)(dstp, srcp, dis, dis, h2ab)
    t2 = _combine2s(p, smooth_alpha_1, smooth_alpha_2)
    p = _sc_smooth(64)(dstp, srcp, dis, dis, t2)
    sm = _combine_mid(p)
    smP = _sc_smooth(32)(dstp, srcp, dis2ad, dis2, sm)
    sm = _combine(smP)
    smP = _sc_smooth(32)(dstp, srcp, dis2ad, dis2, sm)
    sm = _combine(smP)

    return (sm[:N, :30], h4[:N])
